# Initial kernel scaffold; baseline (speedup 1.0000x reference)
import jax
import jax.numpy as jnp
from jax.experimental import pallas as pl


def kernel(semantic_state, token_ids):
    raise NotImplementedError("write your pallas kernel here")


# speedup vs baseline: 3.6097x; 3.6097x over previous
import functools

import jax
import jax.numpy as jnp
from jax import lax
from jax.experimental import pallas as pl
from jax.experimental.pallas import tpu as pltpu
from jax.experimental.pallas import tpu_sc as plsc

_VOCAB = 1000
_VP = 1024
_D = 64
_N = 16384
_NC = 2
_NS = 16
_NW = _NC * _NS
_TPT = _N // _NW
_CH = 128
_NCH = _TPT // _CH
_RPT = _VP // _NS


def _phase1_body(x_hbm, tok_hbm, out_sum, out_sq, out_cnt,
                 idx_v, x_v, sq_v, ones_v, z64, z16,
                 acc_sum, acc_sq, acc_cnt):
    c = lax.axis_index("c")
    s = lax.axis_index("s")
    w = s * _NC + c

    zeros = jnp.zeros((16,), jnp.float32)
    ones = jnp.ones((16,), jnp.float32)

    def zbody(i, carry):
        r = i // 4
        j = (i % 4) * 16
        z64[r, pl.ds(j, 16)] = zeros
        return carry
    lax.fori_loop(0, _RPT * 4, zbody, 0)

    def z16body(r, carry):
        z16[r] = zeros
        return carry
    lax.fori_loop(0, _RPT, z16body, 0)

    def onesbody(r, carry):
        ones_v[r] = ones
        return carry
    lax.fori_loop(0, _TPT, onesbody, 0)

    rows = pl.ds(s * _RPT, _RPT)
    pltpu.sync_copy(z64, acc_sum.at[rows])
    pltpu.sync_copy(z64, acc_sq.at[rows])
    pltpu.sync_copy(z16, acc_cnt.at[rows])

    pltpu.sync_copy(tok_hbm.at[w], idx_v)
    pltpu.sync_copy(x_hbm.at[pl.ds(w * _TPT, _TPT)], x_v)

    def sqbody(i, carry):
        r = i // 4
        j = (i % 4) * 16
        v = x_v[r, pl.ds(j, 16)]
        sq_v[r, pl.ds(j, 16)] = v * v
        return carry
    lax.fori_loop(0, _TPT * 4, sqbody, 0)

    plsc.subcore_barrier()

    for ch in range(_NCH):
        idx = idx_v.at[ch]
        sl = pl.ds(ch * _CH, _CH)
        pltpu.sync_copy(x_v.at[sl], acc_sum.at[idx], add=True)
        pltpu.sync_copy(sq_v.at[sl], acc_sq.at[idx], add=True)
        pltpu.sync_copy(ones_v.at[sl], acc_cnt.at[idx], add=True)

    plsc.subcore_barrier()

    off = c * _VP + s * _RPT
    osl = pl.ds(off, _RPT)
    pltpu.sync_copy(acc_sum.at[rows], out_sum.at[osl])
    pltpu.sync_copy(acc_sq.at[rows], out_sq.at[osl])
    pltpu.sync_copy(acc_cnt.at[rows], out_cnt.at[osl])


_phase1 = functools.partial(
    pl.kernel,
    out_type=(
        jax.ShapeDtypeStruct((_NC * _VP, _D), jnp.float32),
        jax.ShapeDtypeStruct((_NC * _VP, _D), jnp.float32),
        jax.ShapeDtypeStruct((_NC * _VP, 16), jnp.float32),
    ),
    mesh=plsc.VectorSubcoreMesh(
        core_axis_name="c", subcore_axis_name="s",
        num_cores=_NC, num_subcores=_NS),
    scratch_types=[
        pltpu.VMEM((_NCH, _CH), jnp.int32),
        pltpu.VMEM((_TPT, _D), jnp.float32),
        pltpu.VMEM((_TPT, _D), jnp.float32),
        pltpu.VMEM((_TPT, 16), jnp.float32),
        pltpu.VMEM((_RPT, _D), jnp.float32),
        pltpu.VMEM((_RPT, 16), jnp.float32),
        pltpu.VMEM_SHARED((_VP, _D), jnp.float32),
        pltpu.VMEM_SHARED((_VP, _D), jnp.float32),
        pltpu.VMEM_SHARED((_VP, 16), jnp.float32),
    ],
    compiler_params=pltpu.CompilerParams(use_tc_tiling_on_sc=False),
)(_phase1_body)


def _finalize_body(sum_ref, sq_ref, cnt_ref, loss_ref, nrep_ref):
    sums = sum_ref[0] + sum_ref[1]
    sqs = sq_ref[0] + sq_ref[1]
    cnt = cnt_ref[0] + cnt_ref[1]
    c = cnt[:, 0:1]
    mean = sums / jnp.maximum(c, 1.0)
    ss = sqs - c * mean * mean
    var = ss / jnp.maximum(c - 1.0, 1.0)
    var_mean = jnp.sum(var, axis=1, keepdims=True) * (1.0 / _D)
    repeated = c >= 2.0
    nrep = jnp.sum(repeated.astype(jnp.float32))
    total = jnp.sum(jnp.where(repeated, var_mean, 0.0))
    avg = total / jnp.maximum(nrep, 1.0)
    loss = jnp.maximum(1.0 - avg, 0.0)
    loss = jnp.where(nrep > 0.0, loss, 0.0)
    loss_ref[0, 0] = loss
    nrep_ref[0, 0] = nrep.astype(jnp.int32)


_finalize = pl.pallas_call(
    _finalize_body,
    out_shape=(
        jax.ShapeDtypeStruct((1, 1), jnp.float32),
        jax.ShapeDtypeStruct((1, 1), jnp.int32),
    ),
    out_specs=(
        pl.BlockSpec(memory_space=pltpu.SMEM),
        pl.BlockSpec(memory_space=pltpu.SMEM),
    ),
)


@jax.jit
def kernel(semantic_state, token_ids):
    x = semantic_state.reshape(_N, _D)
    tok = token_ids.reshape(_NW, _NCH, _CH).astype(jnp.int32)
    psum, psq, pcnt = _phase1(x, tok)
    loss, nrep = _finalize(
        psum.reshape(_NC, _VP, _D),
        psq.reshape(_NC, _VP, _D),
        pcnt.reshape(_NC, _VP, 16),
    )
    return loss[0, 0], nrep[0, 0]


# speedup vs baseline: 4.3868x; 1.2153x over previous
import functools

import jax
import jax.numpy as jnp
from jax import lax
from jax.experimental import pallas as pl
from jax.experimental.pallas import tpu as pltpu
from jax.experimental.pallas import tpu_sc as plsc

_VOCAB = 1000
_VP = 1024
_D = 64
_B, _T = 4, 4096
_N = _B * _T
_NC = 2
_NS = 16
_NW = _NC * _NS
_TPT = _N // _NW
_WPB = _T // _TPT
_CH = 128
_NCH = _TPT // _CH
_RPT = _VP // _NS


def _phase1_body(x_hbm, tok_hbm, out_sum, out_sq, out_cnt,
                 idx_v, x_v, sq_v, ones_v, z64, z16,
                 acc_sum, acc_sq, acc_cnt,
                 sem_in, sem_z, sem_s, sem_out):
    c = lax.axis_index("c")
    s = lax.axis_index("s")
    w = s * _NC + c
    b = w // _WPB
    t0 = (w % _WPB) * _TPT

    ld_idx = pltpu.async_copy(tok_hbm.at[w], idx_v, sem_in)
    ld_x = pltpu.async_copy(x_hbm.at[b, pl.ds(t0, _TPT)], x_v, sem_in)

    zeros = jnp.zeros((16,), jnp.float32)
    ones = jnp.ones((16,), jnp.float32)

    def zrow(r, carry):
        for j in range(4):
            z64[r, pl.ds(j * 16, 16)] = zeros
        z16[r] = zeros
        return carry
    lax.fori_loop(0, _RPT, zrow, 0)

    def onesrow(r, carry):
        ones_v[r] = ones
        return carry
    lax.fori_loop(0, _CH, onesrow, 0)

    rows = pl.ds(s * _RPT, _RPT)
    z1 = pltpu.async_copy(z64, acc_sum.at[rows], sem_z)
    z2 = pltpu.async_copy(z64, acc_sq.at[rows], sem_z)
    z3 = pltpu.async_copy(z16, acc_cnt.at[rows], sem_z)

    ld_idx.wait()
    ld_x.wait()

    def sqrow(r, carry):
        for j in range(4):
            v = x_v[r, pl.ds(j * 16, 16)]
            sq_v[r, pl.ds(j * 16, 16)] = v * v
        return carry
    lax.fori_loop(0, _TPT, sqrow, 0)

    z1.wait()
    z2.wait()
    z3.wait()
    plsc.subcore_barrier()

    cps = []
    for ch in range(_NCH):
        idx = idx_v.at[ch]
        sl = pl.ds(ch * _CH, _CH)
        cps.append(pltpu.async_copy(x_v.at[sl], acc_sum.at[idx], sem_s, add=True))
        cps.append(pltpu.async_copy(sq_v.at[sl], acc_sq.at[idx], sem_s, add=True))
        cps.append(pltpu.async_copy(ones_v, acc_cnt.at[idx], sem_s, add=True))
    for cp in cps:
        cp.wait()

    plsc.subcore_barrier()

    o1 = pltpu.async_copy(acc_sum.at[rows], out_sum.at[c, s], sem_out)
    o2 = pltpu.async_copy(acc_sq.at[rows], out_sq.at[c, s], sem_out)
    o3 = pltpu.async_copy(acc_cnt.at[rows], out_cnt.at[c, s], sem_out)
    o1.wait()
    o2.wait()
    o3.wait()


_phase1 = functools.partial(
    pl.kernel,
    out_type=(
        jax.ShapeDtypeStruct((_NC, _NS, _RPT, _D), jnp.float32),
        jax.ShapeDtypeStruct((_NC, _NS, _RPT, _D), jnp.float32),
        jax.ShapeDtypeStruct((_NC, _NS, _RPT, 16), jnp.float32),
    ),
    mesh=plsc.VectorSubcoreMesh(
        core_axis_name="c", subcore_axis_name="s",
        num_cores=_NC, num_subcores=_NS),
    scratch_types=[
        pltpu.VMEM((_NCH, _CH), jnp.int32),
        pltpu.VMEM((_TPT, _D), jnp.float32),
        pltpu.VMEM((_TPT, _D), jnp.float32),
        pltpu.VMEM((_CH, 16), jnp.float32),
        pltpu.VMEM((_RPT, _D), jnp.float32),
        pltpu.VMEM((_RPT, 16), jnp.float32),
        pltpu.VMEM_SHARED((_VP, _D), jnp.float32),
        pltpu.VMEM_SHARED((_VP, _D), jnp.float32),
        pltpu.VMEM_SHARED((_VP, 16), jnp.float32),
        pltpu.SemaphoreType.DMA,
        pltpu.SemaphoreType.DMA,
        pltpu.SemaphoreType.DMA,
        pltpu.SemaphoreType.DMA,
    ],
    compiler_params=pltpu.CompilerParams(use_tc_tiling_on_sc=False),
)(_phase1_body)


def _finalize_body(sum_ref, sq_ref, cnt_ref, loss_ref, nrep_ref):
    sums = sum_ref[0] + sum_ref[1]
    sqs = sq_ref[0] + sq_ref[1]
    cnt = cnt_ref[0] + cnt_ref[1]
    c = cnt[:, :, 0:1]
    mean = sums / jnp.maximum(c, 1.0)
    ss = sqs - c * mean * mean
    var = ss / jnp.maximum(c - 1.0, 1.0)
    var_mean = jnp.sum(var, axis=2, keepdims=True) * (1.0 / _D)
    repeated = c >= 2.0
    nrep = jnp.sum(repeated.astype(jnp.float32))
    total = jnp.sum(jnp.where(repeated, var_mean, 0.0))
    avg = total / jnp.maximum(nrep, 1.0)
    loss = jnp.maximum(1.0 - avg, 0.0)
    loss = jnp.where(nrep > 0.0, loss, 0.0)
    loss_ref[0, 0] = loss
    nrep_ref[0, 0] = nrep.astype(jnp.int32)


_finalize = pl.pallas_call(
    _finalize_body,
    out_shape=(
        jax.ShapeDtypeStruct((1, 1), jnp.float32),
        jax.ShapeDtypeStruct((1, 1), jnp.int32),
    ),
    out_specs=(
        pl.BlockSpec(memory_space=pltpu.SMEM),
        pl.BlockSpec(memory_space=pltpu.SMEM),
    ),
)


@jax.jit
def kernel(semantic_state, token_ids):
    tok = token_ids.reshape(_NW, _NCH, _CH).astype(jnp.int32)
    psum, psq, pcnt = _phase1(semantic_state, tok)
    loss, nrep = _finalize(psum, psq, pcnt)
    return loss[0, 0], nrep[0, 0]
